# baseline (speedup 1.0000x reference)
import functools

import jax
import jax.numpy as jnp
from jax import lax
from jax.experimental import pallas as pl
from jax.experimental.pallas import tpu as pltpu
from jax.experimental.pallas import tpu_sc as plsc

K = 1024
DIM = 32
HW = 1024
B = 32
BB = 2
COMMITMENT_COST = 0.25

_NC = 2
_NS = 16
_GROUPS = HW // 16


def _argmin_body(z_ref, emb_ref, idx_ref, loss_ref):
    step = pl.program_id(0)
    emb = emb_ref[...]
    emb_m2 = emb * -2.0
    e2 = jnp.sum(emb * emb, axis=1)
    part = jnp.zeros((1, 1), jnp.float32)
    for i in range(BB):
        z = z_ref[i]
        x2 = jnp.sum(z * z, axis=0)
        dt = jax.lax.dot_general(
            z, emb_m2, (((0,), (1,)), ((), ())),
            preferred_element_type=jnp.float32)
        d = (x2[:, None] + e2[None, :]) + dt
        minv = jnp.min(d, axis=1)
        ciota = jax.lax.broadcasted_iota(jnp.int32, (HW, K), 1)
        is_min = d == minv[:, None]
        idx = jnp.min(jnp.where(is_min, ciota, K), axis=1)
        idx_ref[i] = idx[:, None]
        part = part + jnp.sum(minv).reshape(1, 1)

    @pl.when(step == 0)
    def _():
        loss_ref[...] = part

    @pl.when(step != 0)
    def _():
        loss_ref[...] += part


_SC_MESH = plsc.VectorSubcoreMesh(core_axis_name="c", subcore_axis_name="s")


@functools.partial(
    pl.kernel,
    mesh=_SC_MESH,
    compiler_params=pltpu.CompilerParams(needs_layout_passes=False),
    out_type=jax.ShapeDtypeStruct((B * DIM, HW), jnp.float32),
    scratch_types=[
        pltpu.VMEM((K * DIM,), jnp.float32),
        pltpu.VMEM((HW,), jnp.int32),
        pltpu.VMEM((DIM, HW), jnp.float32),
    ],
)
def _sc_gather(emb_hbm, idx_hbm, out_hbm, emb_v, idx_v, slab_v):
    wid = lax.axis_index("s") * _NC + lax.axis_index("c")
    pltpu.sync_copy(emb_hbm, emb_v)
    pltpu.sync_copy(idx_hbm.at[wid], idx_v)

    @plsc.parallel_loop(0, _GROUPS, unroll=4)
    def _(g):
        pix = idx_v[pl.ds(g * 16, 16)]
        base = pix * DIM
        for c in range(DIM):
            vals = plsc.load_gather(emb_v, [base + c])
            slab_v[c, pl.ds(g * 16, 16)] = vals
    pltpu.sync_copy(slab_v, out_hbm.at[pl.ds(wid * DIM, DIM)])


def kernel(z_e, emb_weight):
    z3 = z_e.reshape(B, DIM, HW)
    idx3, loss_raw = pl.pallas_call(
        _argmin_body,
        grid=(B // BB,),
        in_specs=[
            pl.BlockSpec((BB, DIM, HW), lambda b: (b, 0, 0)),
            pl.BlockSpec((K, DIM), lambda b: (0, 0)),
        ],
        out_specs=[
            pl.BlockSpec((BB, HW, 1), lambda b: (b, 0, 0)),
            pl.BlockSpec((1, 1), lambda b: (0, 0)),
        ],
        out_shape=[
            jax.ShapeDtypeStruct((B, HW, 1), jnp.int32),
            jax.ShapeDtypeStruct((1, 1), jnp.float32),
        ],
    )(z3, emb_weight)
    zq2 = _sc_gather(emb_weight.reshape(K * DIM), idx3.reshape(B, HW))
    z_q_st = zq2.reshape(z_e.shape)
    loss = loss_raw[0, 0] * ((1.0 + COMMITMENT_COST) / (B * DIM * HW))
    return (z_q_st, loss)


# speedup vs baseline: 1.6281x; 1.6281x over previous
import jax
import jax.numpy as jnp
from jax.experimental import pallas as pl

K = 1024
DIM = 32
HW = 1024
B = 32
BB = 2
COMMITMENT_COST = 0.25


def _vq_body(z_ref, emb_ref, zq_ref, loss_ref):
    step = pl.program_id(0)
    emb = emb_ref[...]
    emb_m2 = emb * -2.0
    e2 = jnp.sum(emb * emb, axis=1)
    part = jnp.zeros((1, 1), jnp.float32)
    for i in range(BB):
        z = z_ref[i].reshape(DIM, HW)
        x2 = jnp.sum(z * z, axis=0)
        dt = jax.lax.dot_general(
            z, emb_m2, (((0,), (1,)), ((), ())),
            preferred_element_type=jnp.float32)
        d = (x2[:, None] + e2[None, :]) + dt
        minv = jnp.min(d, axis=1)
        ciota = jax.lax.broadcasted_iota(jnp.int32, (HW, K), 1)
        is_min = d == minv[:, None]
        idx = jnp.min(jnp.where(is_min, ciota, K), axis=1)
        onehot = (idx[:, None] == ciota).astype(jnp.bfloat16)
        zq = jax.lax.dot_general(
            emb, onehot, (((0,), (1,)), ((), ())),
            preferred_element_type=jnp.float32)
        zq_ref[i] = (z + (zq - z)).reshape(DIM, 32, 32)
        part = part + jnp.sum(minv).reshape(1, 1)

    @pl.when(step == 0)
    def _():
        loss_ref[...] = part

    @pl.when(step != 0)
    def _():
        loss_ref[...] += part


def kernel(z_e, emb_weight):
    z_q_st, loss_raw = pl.pallas_call(
        _vq_body,
        grid=(B // BB,),
        in_specs=[
            pl.BlockSpec((BB, DIM, 32, 32), lambda b: (b, 0, 0, 0)),
            pl.BlockSpec((K, DIM), lambda b: (0, 0)),
        ],
        out_specs=[
            pl.BlockSpec((BB, DIM, 32, 32), lambda b: (b, 0, 0, 0)),
            pl.BlockSpec((1, 1), lambda b: (0, 0)),
        ],
        out_shape=[
            jax.ShapeDtypeStruct((B, DIM, 32, 32), jnp.float32),
            jax.ShapeDtypeStruct((1, 1), jnp.float32),
        ],
    )(z_e, emb_weight)
    loss = loss_raw[0, 0] * ((1.0 + COMMITMENT_COST) / (B * DIM * HW))
    return (z_q_st, loss)
